# baseline (speedup 1.0000x reference)
import functools

import jax
import jax.numpy as jnp
from jax import lax
from jax.experimental import pallas as pl
from jax.experimental.pallas import tpu as pltpu
from jax.experimental.pallas import tpu_sc as plsc

D_MODEL = 1024
D_FF = 4096
E = 8
T = 8192

TB = 1024
B = 256
MAX_BLOCKS = T // B + E - 1
CAPB = T // B
PARK = E * CAPB
XS_ROWS = E * T + B

NC, NS = 2, 16
NW = NC * NS
TPW = T // NW
CHUNK = 32
NCHUNK = TPW // CHUNK


def _router_body(x_ref, rw_ref, w16_ref, dst_ref, xblk_ref, carry_ref):
    g = pl.program_id(0)

    @pl.when(g == 0)
    def _():
        carry_ref[...] = jnp.zeros_like(carry_ref)

    x = x_ref[...]
    logits = lax.dot_general(x, rw_ref[...],
                             (((1,), (1,)), ((), ())),
                             preferred_element_type=jnp.float32)
    m = jnp.max(logits, axis=1, keepdims=True)
    s = jnp.sum(jnp.exp(logits - m), axis=1, keepdims=True)
    w = 1.0 / s
    w16_ref[...] = jnp.broadcast_to(w, (TB, 128))

    e_iota = lax.broadcasted_iota(jnp.int32, (TB, E), 1)
    idx = jnp.min(jnp.where(logits == m, e_iota, E), axis=1)

    oh = (lax.broadcasted_iota(jnp.int32, (E, TB), 0)
          == idx[None, :]).astype(jnp.float32)
    tl = (lax.broadcasted_iota(jnp.int32, (TB, TB), 0)
          < lax.broadcasted_iota(jnp.int32, (TB, TB), 1)).astype(jnp.float32)
    within = lax.dot_general(oh, tl, (((1,), (0,)), ((), ())),
                             preferred_element_type=jnp.float32)
    carry = carry_ref[...][:, 0:1]
    rank = jnp.sum(oh * (within + carry), axis=0)
    dst_ref[...] = idx * T + rank.astype(jnp.int32)

    new_carry = carry + jnp.sum(oh, axis=1, keepdims=True)
    carry_ref[...] = jnp.broadcast_to(new_carry, (E, 128))

    counts = new_carry.astype(jnp.int32)
    nb = lax.shift_right_logical(counts + (B - 1), B.bit_length() - 1)
    tli = (lax.broadcasted_iota(jnp.int32, (E, E), 0)
           <= lax.broadcasted_iota(jnp.int32, (E, E), 1)).astype(jnp.float32)
    end_b = lax.dot_general(nb.astype(jnp.float32)[:, 0][None, :], tli,
                            (((1,), (0,)), ((), ())),
                            preferred_element_type=jnp.float32)
    end_b = end_b.astype(jnp.int32).reshape(E, 1)
    start_b = end_b - nb
    g_iota = lax.broadcasted_iota(jnp.int32, (E, 128), 1)
    eg = jnp.minimum(jnp.sum((g_iota >= end_b).astype(jnp.int32), axis=0),
                     E - 1)
    ohg = (lax.broadcasted_iota(jnp.int32, (E, 128), 0) == eg[None, :])
    startg = jnp.sum(jnp.where(ohg, start_b, 0), axis=0)
    gi = g_iota[0]
    xblk = eg * CAPB + (gi - startg)
    total = jnp.max(end_b, axis=0)
    xblk_ref[...] = jnp.where(gi < total, xblk, PARK)


def _run_router(x, router_w):
    return pl.pallas_call(
        _router_body,
        grid=(T // TB,),
        in_specs=[
            pl.BlockSpec((TB, D_MODEL), lambda g: (g, 0)),
            pl.BlockSpec((E, D_MODEL), lambda g: (0, 0)),
        ],
        out_specs=[
            pl.BlockSpec((TB, 128), lambda g: (g, 0)),
            pl.BlockSpec((TB,), lambda g: (g,)),
            pl.BlockSpec((128,), lambda g: (0,)),
        ],
        out_shape=[
            jax.ShapeDtypeStruct((T, 128), jnp.float32),
            jax.ShapeDtypeStruct((T,), jnp.int32),
            jax.ShapeDtypeStruct((128,), jnp.int32),
        ],
        scratch_shapes=[pltpu.VMEM((E, 128), jnp.float32)],
    )(x, router_w)


def _sc_scatter(x, w16, dst2d):
    mesh = plsc.VectorSubcoreMesh(core_axis_name="c", subcore_axis_name="s")

    @functools.partial(
        pl.kernel,
        mesh=mesh,
        out_type=(
            jax.ShapeDtypeStruct((XS_ROWS, D_MODEL), jnp.float32),
            jax.ShapeDtypeStruct((XS_ROWS, 128), jnp.float32),
        ),
        scratch_types=[
            pltpu.VMEM((NCHUNK, CHUNK), jnp.int32),
            pltpu.VMEM((2, CHUNK, D_MODEL), jnp.float32),
            pltpu.VMEM((2, CHUNK, 128), jnp.float32),
            pltpu.SemaphoreType.DMA,
            pltpu.SemaphoreType.DMA,
            pltpu.SemaphoreType.DMA,
            pltpu.SemaphoreType.DMA,
        ],
    )
    def k(x_hbm, w16_hbm, dst_hbm, xs_hbm, ws_hbm, idx_v, rows_v, wrows_v,
          sem_lx, sem_sx, sem_lw, sem_sw):
        wid = lax.axis_index("s") * NC + lax.axis_index("c")
        pltpu.sync_copy(dst_hbm.at[pl.ds(wid * NCHUNK, NCHUNK)], idx_v)
        base = wid * TPW

        def lx(c):
            return pltpu.make_async_copy(
                x_hbm.at[pl.ds(base + c * CHUNK, CHUNK)], rows_v.at[c % 2],
                sem_lx)

        def sx(c):
            return pltpu.make_async_copy(
                rows_v.at[c % 2], xs_hbm.at[idx_v.at[c]], sem_sx)

        def lw(c):
            return pltpu.make_async_copy(
                w16_hbm.at[pl.ds(base + c * CHUNK, CHUNK)],
                wrows_v.at[c % 2], sem_lw)

        def sw(c):
            return pltpu.make_async_copy(
                wrows_v.at[c % 2], ws_hbm.at[idx_v.at[c]], sem_sw)

        lx(0).start()
        lw(0).start()
        for c in range(NCHUNK):
            lx(c).wait()
            lw(c).wait()
            if c + 1 < NCHUNK:
                if c >= 1:
                    sx(c - 1).wait()
                    sw(c - 1).wait()
                lx(c + 1).start()
                lw(c + 1).start()
            sx(c).start()
            sw(c).start()
        sx(NCHUNK - 2).wait()
        sw(NCHUNK - 2).wait()
        sx(NCHUNK - 1).wait()
        sw(NCHUNK - 1).wait()

    return k(x, w16, dst2d)


def _blk_expert(xblk):
    return jnp.minimum(lax.shift_right_logical(xblk, CAPB.bit_length() - 1),
                       E - 1)


def _ffn_body(be_ref, x_ref, w1_hbm, w2_hbm, ws_ref, y_ref,
              w1_v, w2_v, slot_ref, sem1, sem2):
    g = pl.program_id(0)
    e = _blk_expert(be_ref[g])
    switched = (g == 0) | (e != _blk_expert(be_ref[jnp.maximum(g - 1, 0)]))

    @pl.when(g == 0)
    def _():
        slot_ref[0] = 0
        pltpu.make_async_copy(w1_hbm.at[e], w1_v.at[0], sem1).start()
        pltpu.make_async_copy(w2_hbm.at[e], w2_v, sem2).start()

    @pl.when((g > 0) & switched)
    def _():
        slot_ref[0] = 1 - slot_ref[0]

    s = slot_ref[0]

    @pl.when(switched)
    def _():
        pltpu.make_async_copy(w1_hbm.at[0], w1_v.at[s], sem1).wait()
        j = lax.while_loop(
            lambda j: (j < MAX_BLOCKS)
            & (_blk_expert(be_ref[jnp.minimum(j, MAX_BLOCKS - 1)]) == e),
            lambda j: j + 1, g + 1)

        @pl.when(j < MAX_BLOCKS)
        def _():
            e_nd = _blk_expert(be_ref[jnp.minimum(j, MAX_BLOCKS - 1)])
            pltpu.make_async_copy(w1_hbm.at[e_nd], w1_v.at[1 - s],
                                  sem1).start()

    x = x_ref[...]
    h = lax.dot_general(x, w1_v[s], (((1,), (1,)), ((), ())),
                        preferred_element_type=jnp.float32)
    h = h * (1.0 / (1.0 + jnp.exp(-h)))

    @pl.when(switched)
    def _():
        pltpu.make_async_copy(w2_hbm.at[0], w2_v, sem2).wait()

    y = lax.dot_general(h, w2_v[...], (((1,), (1,)), ((), ())),
                        preferred_element_type=jnp.float32)
    y_ref[...] = y * ws_ref[...][:, 0:1]

    e_next2 = _blk_expert(be_ref[jnp.minimum(g + 1, MAX_BLOCKS - 1)])

    @pl.when((g + 1 < MAX_BLOCKS) & (e_next2 != e))
    def _():
        pltpu.make_async_copy(w2_hbm.at[e_next2], w2_v, sem2).start()


def _run_ffn(block_expert, x_sorted, w1, w2, w_sorted):
    grid_spec = pltpu.PrefetchScalarGridSpec(
        num_scalar_prefetch=1,
        grid=(MAX_BLOCKS,),
        in_specs=[
            pl.BlockSpec((B, D_MODEL), lambda g, be: (be[g], 0)),
            pl.BlockSpec(memory_space=pl.ANY),
            pl.BlockSpec(memory_space=pl.ANY),
            pl.BlockSpec((B, 128), lambda g, be: (be[g], 0)),
        ],
        out_specs=pl.BlockSpec((B, D_MODEL), lambda g, be: (be[g], 0)),
        scratch_shapes=[
            pltpu.VMEM((2, D_FF, D_MODEL), jnp.float32),
            pltpu.VMEM((D_MODEL, D_FF), jnp.float32),
            pltpu.SMEM((1,), jnp.int32),
            pltpu.SemaphoreType.DMA,
            pltpu.SemaphoreType.DMA,
        ],
    )
    return pl.pallas_call(
        _ffn_body,
        grid_spec=grid_spec,
        out_shape=jax.ShapeDtypeStruct((XS_ROWS, D_MODEL), jnp.float32),
        compiler_params=pltpu.CompilerParams(
            vmem_limit_bytes=100 * 1024 * 1024),
    )(block_expert, x_sorted, w1, w2, w_sorted)


def _sc_gather(y_sorted, dst2d):
    mesh = plsc.VectorSubcoreMesh(core_axis_name="c", subcore_axis_name="s")

    @functools.partial(
        pl.kernel,
        mesh=mesh,
        out_type=jax.ShapeDtypeStruct((T, D_MODEL), jnp.float32),
        scratch_types=[
            pltpu.VMEM((NCHUNK, CHUNK), jnp.int32),
            pltpu.VMEM((2, CHUNK, D_MODEL), jnp.float32),
            pltpu.SemaphoreType.DMA,
            pltpu.SemaphoreType.DMA,
        ],
    )
    def k(ys_hbm, dst_hbm, out_hbm, idx_v, rows_v, sem_g, sem_s):
        wid = lax.axis_index("s") * NC + lax.axis_index("c")
        pltpu.sync_copy(dst_hbm.at[pl.ds(wid * NCHUNK, NCHUNK)], idx_v)
        base = wid * TPW

        def gy(c):
            return pltpu.make_async_copy(
                ys_hbm.at[idx_v.at[c]], rows_v.at[c % 2], sem_g)

        def st(c):
            return pltpu.make_async_copy(
                rows_v.at[c % 2], out_hbm.at[pl.ds(base + c * CHUNK, CHUNK)],
                sem_s)

        gy(0).start()
        for c in range(NCHUNK):
            gy(c).wait()
            if c + 1 < NCHUNK:
                if c >= 1:
                    st(c - 1).wait()
                gy(c + 1).start()
            st(c).start()
        st(NCHUNK - 2).wait()
        st(NCHUNK - 1).wait()

    return k(y_sorted, dst2d)


def kernel(x, router_w, w1, w2):
    w16, dst, xblk = _run_router(x, router_w)
    dst2d = dst.reshape(T // CHUNK, CHUNK)
    x_sorted, w_sorted = _sc_scatter(x, w16, dst2d)
    y_sorted = _run_ffn(xblk, x_sorted, w1, w2, w_sorted)
    return _sc_gather(y_sorted, dst2d)


# speedup vs baseline: 1.0488x; 1.0488x over previous
import functools

import jax
import jax.numpy as jnp
from jax import lax
from jax.experimental import pallas as pl
from jax.experimental.pallas import tpu as pltpu
from jax.experimental.pallas import tpu_sc as plsc

D_MODEL = 1024
D_FF = 4096
E = 8
T = 8192

TB = 1024
B = 256
MAX_BLOCKS = T // B + E - 1
CAPB = T // B
PARK = E * CAPB
XS_ROWS = E * T + B

NC, NS = 2, 16
NW = NC * NS
TPW = T // NW
CHUNK = 32
NCHUNK = TPW // CHUNK


def _router_body(x_ref, rw_ref, w16_ref, dst_ref, xblk_ref, carry_ref):
    g = pl.program_id(0)

    @pl.when(g == 0)
    def _():
        carry_ref[...] = jnp.zeros_like(carry_ref)

    x = x_ref[...]
    logits = lax.dot_general(x, rw_ref[...],
                             (((1,), (1,)), ((), ())),
                             preferred_element_type=jnp.float32)
    m = jnp.max(logits, axis=1, keepdims=True)
    s = jnp.sum(jnp.exp(logits - m), axis=1, keepdims=True)
    w = 1.0 / s
    w16_ref[...] = jnp.broadcast_to(w, (TB, 128))

    e_iota = lax.broadcasted_iota(jnp.int32, (TB, E), 1)
    idx = jnp.min(jnp.where(logits == m, e_iota, E), axis=1)

    oh = (lax.broadcasted_iota(jnp.int32, (E, TB), 0)
          == idx[None, :]).astype(jnp.float32)
    tl = (lax.broadcasted_iota(jnp.int32, (TB, TB), 0)
          < lax.broadcasted_iota(jnp.int32, (TB, TB), 1)).astype(jnp.float32)
    within = lax.dot_general(oh, tl, (((1,), (0,)), ((), ())),
                             preferred_element_type=jnp.float32)
    carry = carry_ref[...][:, 0:1]
    rank = jnp.sum(oh * (within + carry), axis=0)
    dst_ref[...] = idx * T + rank.astype(jnp.int32)

    new_carry = carry + jnp.sum(oh, axis=1, keepdims=True)
    carry_ref[...] = jnp.broadcast_to(new_carry, (E, 128))

    counts = new_carry.astype(jnp.int32)
    nb = lax.shift_right_logical(counts + (B - 1), B.bit_length() - 1)
    tli = (lax.broadcasted_iota(jnp.int32, (E, E), 0)
           <= lax.broadcasted_iota(jnp.int32, (E, E), 1)).astype(jnp.float32)
    end_b = lax.dot_general(nb.astype(jnp.float32)[:, 0][None, :], tli,
                            (((1,), (0,)), ((), ())),
                            preferred_element_type=jnp.float32)
    end_b = end_b.astype(jnp.int32).reshape(E, 1)
    start_b = end_b - nb
    g_iota = lax.broadcasted_iota(jnp.int32, (E, 128), 1)
    eg = jnp.minimum(jnp.sum((g_iota >= end_b).astype(jnp.int32), axis=0),
                     E - 1)
    ohg = (lax.broadcasted_iota(jnp.int32, (E, 128), 0) == eg[None, :])
    startg = jnp.sum(jnp.where(ohg, start_b, 0), axis=0)
    gi = g_iota[0]
    xblk = eg * CAPB + (gi - startg)
    total = jnp.max(end_b, axis=0)
    xblk_ref[...] = jnp.where(gi < total, xblk, PARK)


def _run_router(x, router_w):
    return pl.pallas_call(
        _router_body,
        grid=(T // TB,),
        in_specs=[
            pl.BlockSpec((TB, D_MODEL), lambda g: (g, 0)),
            pl.BlockSpec((E, D_MODEL), lambda g: (0, 0)),
        ],
        out_specs=[
            pl.BlockSpec((TB, 128), lambda g: (g, 0)),
            pl.BlockSpec((TB,), lambda g: (g,)),
            pl.BlockSpec((128,), lambda g: (0,)),
        ],
        out_shape=[
            jax.ShapeDtypeStruct((T, 128), jnp.float32),
            jax.ShapeDtypeStruct((T,), jnp.int32),
            jax.ShapeDtypeStruct((128,), jnp.int32),
        ],
        scratch_shapes=[pltpu.VMEM((E, 128), jnp.float32)],
    )(x, router_w)


def _sc_scatter(x, w16, dst2d):
    mesh = plsc.VectorSubcoreMesh(core_axis_name="c", subcore_axis_name="s")

    @functools.partial(
        pl.kernel,
        mesh=mesh,
        out_type=(
            jax.ShapeDtypeStruct((XS_ROWS, D_MODEL), jnp.float32),
            jax.ShapeDtypeStruct((XS_ROWS, 128), jnp.float32),
        ),
        scratch_types=[
            pltpu.VMEM((NCHUNK, CHUNK), jnp.int32),
            pltpu.VMEM((3, CHUNK, D_MODEL), jnp.float32),
            pltpu.VMEM((3, CHUNK, 128), jnp.float32),
            pltpu.SemaphoreType.DMA,
            pltpu.SemaphoreType.DMA,
            pltpu.SemaphoreType.DMA,
            pltpu.SemaphoreType.DMA,
        ],
    )
    def k(x_hbm, w16_hbm, dst_hbm, xs_hbm, ws_hbm, idx_v, rows_v, wrows_v,
          sem_lx, sem_sx, sem_lw, sem_sw):
        wid = lax.axis_index("s") * NC + lax.axis_index("c")
        pltpu.sync_copy(dst_hbm.at[pl.ds(wid * NCHUNK, NCHUNK)], idx_v)
        base = wid * TPW

        def lx(c):
            return pltpu.make_async_copy(
                x_hbm.at[pl.ds(base + c * CHUNK, CHUNK)], rows_v.at[c % 3],
                sem_lx)

        def sx(c):
            return pltpu.make_async_copy(
                rows_v.at[c % 3], xs_hbm.at[idx_v.at[c]], sem_sx)

        def lw(c):
            return pltpu.make_async_copy(
                w16_hbm.at[pl.ds(base + c * CHUNK, CHUNK)],
                wrows_v.at[c % 3], sem_lw)

        def sw(c):
            return pltpu.make_async_copy(
                wrows_v.at[c % 3], ws_hbm.at[idx_v.at[c]], sem_sw)

        lx(0).start()
        lw(0).start()
        lx(1).start()
        lw(1).start()
        for c in range(NCHUNK):
            lx(c).wait()
            lw(c).wait()
            if c + 2 < NCHUNK:
                if c >= 1:
                    sx(c - 1).wait()
                    sw(c - 1).wait()
                lx(c + 2).start()
                lw(c + 2).start()
            sx(c).start()
            sw(c).start()
        for c in range(NCHUNK - 3, NCHUNK):
            sx(c).wait()
            sw(c).wait()

    return k(x, w16, dst2d)


def _blk_expert(xblk):
    return jnp.minimum(lax.shift_right_logical(xblk, CAPB.bit_length() - 1),
                       E - 1)


def _ffn_body(be_ref, x_ref, w1_hbm, w2_hbm, ws_ref, y_ref,
              w1_v, w2_v, slot_ref, sem1, sem2):
    g = pl.program_id(0)
    blk = be_ref[g]
    active = blk != PARK
    e = _blk_expert(blk)
    switched = active & (
        (g == 0) | (e != _blk_expert(be_ref[jnp.maximum(g - 1, 0)])))

    @pl.when(g == 0)
    def _():
        slot_ref[0] = 0
        pltpu.make_async_copy(w1_hbm.at[e], w1_v.at[0], sem1).start()
        pltpu.make_async_copy(w2_hbm.at[e], w2_v, sem2).start()

    @pl.when((g > 0) & switched)
    def _():
        slot_ref[0] = 1 - slot_ref[0]

    s = slot_ref[0]

    @pl.when(switched)
    def _():
        pltpu.make_async_copy(w1_hbm.at[0], w1_v.at[s], sem1).wait()
        j = lax.while_loop(
            lambda j: (j < MAX_BLOCKS)
            & (_blk_expert(be_ref[jnp.minimum(j, MAX_BLOCKS - 1)]) == e),
            lambda j: j + 1, g + 1)
        jc = jnp.minimum(j, MAX_BLOCKS - 1)

        @pl.when((j < MAX_BLOCKS) & (be_ref[jc] != PARK))
        def _():
            pltpu.make_async_copy(w1_hbm.at[_blk_expert(be_ref[jc])],
                                  w1_v.at[1 - s], sem1).start()

    @pl.when(active)
    def _():
        x = x_ref[...]
        h = lax.dot_general(x, w1_v[s], (((1,), (1,)), ((), ())),
                            preferred_element_type=jnp.float32)
        h = h * (1.0 / (1.0 + jnp.exp(-h)))

        @pl.when(switched)
        def _():
            pltpu.make_async_copy(w2_hbm.at[0], w2_v, sem2).wait()

        y = lax.dot_general(h, w2_v[...], (((1,), (1,)), ((), ())),
                            preferred_element_type=jnp.float32)
        y_ref[...] = y * ws_ref[...][:, 0:1]

        nblk = be_ref[jnp.minimum(g + 1, MAX_BLOCKS - 1)]
        e_next2 = _blk_expert(nblk)

        @pl.when((g + 1 < MAX_BLOCKS) & (e_next2 != e) & (nblk != PARK))
        def _():
            pltpu.make_async_copy(w2_hbm.at[e_next2], w2_v, sem2).start()


def _run_ffn(block_expert, x_sorted, w1, w2, w_sorted):
    grid_spec = pltpu.PrefetchScalarGridSpec(
        num_scalar_prefetch=1,
        grid=(MAX_BLOCKS,),
        in_specs=[
            pl.BlockSpec((B, D_MODEL), lambda g, be: (be[g], 0)),
            pl.BlockSpec(memory_space=pl.ANY),
            pl.BlockSpec(memory_space=pl.ANY),
            pl.BlockSpec((B, 128), lambda g, be: (be[g], 0)),
        ],
        out_specs=pl.BlockSpec((B, D_MODEL), lambda g, be: (be[g], 0)),
        scratch_shapes=[
            pltpu.VMEM((2, D_FF, D_MODEL), jnp.float32),
            pltpu.VMEM((D_MODEL, D_FF), jnp.float32),
            pltpu.SMEM((1,), jnp.int32),
            pltpu.SemaphoreType.DMA,
            pltpu.SemaphoreType.DMA,
        ],
    )
    return pl.pallas_call(
        _ffn_body,
        grid_spec=grid_spec,
        out_shape=jax.ShapeDtypeStruct((XS_ROWS, D_MODEL), jnp.float32),
        compiler_params=pltpu.CompilerParams(
            vmem_limit_bytes=100 * 1024 * 1024),
    )(block_expert, x_sorted, w1, w2, w_sorted)


def _sc_gather(y_sorted, dst2d):
    mesh = plsc.VectorSubcoreMesh(core_axis_name="c", subcore_axis_name="s")

    @functools.partial(
        pl.kernel,
        mesh=mesh,
        out_type=jax.ShapeDtypeStruct((T, D_MODEL), jnp.float32),
        scratch_types=[
            pltpu.VMEM((NCHUNK, CHUNK), jnp.int32),
            pltpu.VMEM((3, CHUNK, D_MODEL), jnp.float32),
            pltpu.SemaphoreType.DMA,
            pltpu.SemaphoreType.DMA,
        ],
    )
    def k(ys_hbm, dst_hbm, out_hbm, idx_v, rows_v, sem_g, sem_s):
        wid = lax.axis_index("s") * NC + lax.axis_index("c")
        pltpu.sync_copy(dst_hbm.at[pl.ds(wid * NCHUNK, NCHUNK)], idx_v)
        base = wid * TPW

        def gy(c):
            return pltpu.make_async_copy(
                ys_hbm.at[idx_v.at[c]], rows_v.at[c % 3], sem_g)

        def st(c):
            return pltpu.make_async_copy(
                rows_v.at[c % 3], out_hbm.at[pl.ds(base + c * CHUNK, CHUNK)],
                sem_s)

        gy(0).start()
        gy(1).start()
        for c in range(NCHUNK):
            gy(c).wait()
            if c + 2 < NCHUNK:
                if c >= 1:
                    st(c - 1).wait()
                gy(c + 2).start()
            st(c).start()
        for c in range(NCHUNK - 3, NCHUNK):
            st(c).wait()

    return k(y_sorted, dst2d)


def kernel(x, router_w, w1, w2):
    w16, dst, xblk = _run_router(x, router_w)
    dst2d = dst.reshape(T // CHUNK, CHUNK)
    x_sorted, w_sorted = _sc_scatter(x, w16, dst2d)
    y_sorted = _run_ffn(xblk, x_sorted, w1, w2, w_sorted)
    return _sc_gather(y_sorted, dst2d)


# speedup vs baseline: 1.0663x; 1.0167x over previous
import functools

import jax
import jax.numpy as jnp
from jax import lax
from jax.experimental import pallas as pl
from jax.experimental.pallas import tpu as pltpu
from jax.experimental.pallas import tpu_sc as plsc

D_MODEL = 1024
D_FF = 4096
E = 8
T = 8192

TB = 1024
B = 256
MAX_BLOCKS = T // B + E - 1
CAPB = T // B
PARK = E * CAPB
XS_ROWS = E * T + B

NC, NS = 2, 16
NW = NC * NS
TPW = T // NW
CHUNK = 32
NCHUNK = TPW // CHUNK


def _router_body(x_ref, rw_ref, dst_ref, xblk_ref, carry_ref):
    g = pl.program_id(0)

    @pl.when(g == 0)
    def _():
        carry_ref[...] = jnp.zeros_like(carry_ref)

    x = x_ref[...]
    logits = lax.dot_general(x, rw_ref[...],
                             (((1,), (1,)), ((), ())),
                             preferred_element_type=jnp.float32)
    m = jnp.max(logits, axis=1, keepdims=True)
    e_iota = lax.broadcasted_iota(jnp.int32, (TB, E), 1)
    idx = jnp.min(jnp.where(logits == m, e_iota, E), axis=1)

    oh = (lax.broadcasted_iota(jnp.int32, (E, TB), 0)
          == idx[None, :]).astype(jnp.float32)
    tl = (lax.broadcasted_iota(jnp.int32, (TB, TB), 0)
          < lax.broadcasted_iota(jnp.int32, (TB, TB), 1)).astype(jnp.float32)
    within = lax.dot_general(oh, tl, (((1,), (0,)), ((), ())),
                             preferred_element_type=jnp.float32)
    carry = carry_ref[...][:, 0:1]
    rank = jnp.sum(oh * (within + carry), axis=0)
    dst_ref[...] = idx * T + rank.astype(jnp.int32)

    new_carry = carry + jnp.sum(oh, axis=1, keepdims=True)
    carry_ref[...] = jnp.broadcast_to(new_carry, (E, 128))

    counts = new_carry.astype(jnp.int32)
    nb = lax.shift_right_logical(counts + (B - 1), B.bit_length() - 1)
    tli = (lax.broadcasted_iota(jnp.int32, (E, E), 0)
           <= lax.broadcasted_iota(jnp.int32, (E, E), 1)).astype(jnp.float32)
    end_b = lax.dot_general(nb.astype(jnp.float32)[:, 0][None, :], tli,
                            (((1,), (0,)), ((), ())),
                            preferred_element_type=jnp.float32)
    end_b = end_b.astype(jnp.int32).reshape(E, 1)
    start_b = end_b - nb
    g_iota = lax.broadcasted_iota(jnp.int32, (E, 128), 1)
    eg = jnp.minimum(jnp.sum((g_iota >= end_b).astype(jnp.int32), axis=0),
                     E - 1)
    ohg = (lax.broadcasted_iota(jnp.int32, (E, 128), 0) == eg[None, :])
    startg = jnp.sum(jnp.where(ohg, start_b, 0), axis=0)
    gi = g_iota[0]
    xblk = eg * CAPB + (gi - startg)
    total = jnp.max(end_b, axis=0)
    xblk_ref[...] = jnp.where(gi < total, xblk, PARK)


def _run_router(x, router_w):
    return pl.pallas_call(
        _router_body,
        grid=(T // TB,),
        in_specs=[
            pl.BlockSpec((TB, D_MODEL), lambda g: (g, 0)),
            pl.BlockSpec((E, D_MODEL), lambda g: (0, 0)),
        ],
        out_specs=[
            pl.BlockSpec((TB,), lambda g: (g,)),
            pl.BlockSpec((128,), lambda g: (0,)),
        ],
        out_shape=[
            jax.ShapeDtypeStruct((T,), jnp.int32),
            jax.ShapeDtypeStruct((128,), jnp.int32),
        ],
        scratch_shapes=[pltpu.VMEM((E, 128), jnp.float32)],
    )(x, router_w)


def _sc_scatter(x, dst2d):
    mesh = plsc.VectorSubcoreMesh(core_axis_name="c", subcore_axis_name="s")

    @functools.partial(
        pl.kernel,
        mesh=mesh,
        out_type=jax.ShapeDtypeStruct((XS_ROWS, D_MODEL), jnp.float32),
        scratch_types=[
            pltpu.VMEM((NCHUNK, CHUNK), jnp.int32),
            pltpu.VMEM((3, CHUNK, D_MODEL), jnp.float32),
            pltpu.SemaphoreType.DMA,
            pltpu.SemaphoreType.DMA,
        ],
    )
    def k(x_hbm, dst_hbm, xs_hbm, idx_v, rows_v, sem_lx, sem_sx):
        wid = lax.axis_index("s") * NC + lax.axis_index("c")
        pltpu.sync_copy(dst_hbm.at[pl.ds(wid * NCHUNK, NCHUNK)], idx_v)
        base = wid * TPW

        def lx(c):
            return pltpu.make_async_copy(
                x_hbm.at[pl.ds(base + c * CHUNK, CHUNK)], rows_v.at[c % 3],
                sem_lx)

        def sx(c):
            return pltpu.make_async_copy(
                rows_v.at[c % 3], xs_hbm.at[idx_v.at[c]], sem_sx)

        lx(0).start()
        lx(1).start()
        for c in range(NCHUNK):
            lx(c).wait()
            if c + 2 < NCHUNK:
                if c >= 1:
                    sx(c - 1).wait()
                lx(c + 2).start()
            sx(c).start()
        for c in range(NCHUNK - 3, NCHUNK):
            sx(c).wait()

    return k(x, dst2d)


def _blk_expert(xblk):
    return jnp.minimum(lax.shift_right_logical(xblk, CAPB.bit_length() - 1),
                       E - 1)


def _ffn_body(be_ref, x_ref, w1_hbm, w2_hbm, rw_ref, y_ref,
              w1_v, w2_v, slot_ref, sem1, sem2):
    g = pl.program_id(0)
    blk = be_ref[g]
    active = blk != PARK
    e = _blk_expert(blk)
    switched = active & (
        (g == 0) | (e != _blk_expert(be_ref[jnp.maximum(g - 1, 0)])))

    @pl.when(g == 0)
    def _():
        slot_ref[0] = 0
        pltpu.make_async_copy(w1_hbm.at[e], w1_v.at[0], sem1).start()
        pltpu.make_async_copy(w2_hbm.at[e], w2_v, sem2).start()

    @pl.when((g > 0) & switched)
    def _():
        slot_ref[0] = 1 - slot_ref[0]

    s = slot_ref[0]

    @pl.when(switched)
    def _():
        pltpu.make_async_copy(w1_hbm.at[0], w1_v.at[s], sem1).wait()
        j = lax.while_loop(
            lambda j: (j < MAX_BLOCKS)
            & (_blk_expert(be_ref[jnp.minimum(j, MAX_BLOCKS - 1)]) == e),
            lambda j: j + 1, g + 1)
        jc = jnp.minimum(j, MAX_BLOCKS - 1)

        @pl.when((j < MAX_BLOCKS) & (be_ref[jc] != PARK))
        def _():
            pltpu.make_async_copy(w1_hbm.at[_blk_expert(be_ref[jc])],
                                  w1_v.at[1 - s], sem1).start()

    @pl.when(active)
    def _():
        x = x_ref[...]
        logits = lax.dot_general(x, rw_ref[...], (((1,), (1,)), ((), ())),
                                 preferred_element_type=jnp.float32)
        m = jnp.max(logits, axis=1, keepdims=True)
        w = 1.0 / jnp.sum(jnp.exp(logits - m), axis=1, keepdims=True)

        h = lax.dot_general(x, w1_v[s], (((1,), (1,)), ((), ())),
                            preferred_element_type=jnp.float32)
        h = h * (1.0 / (1.0 + jnp.exp(-h)))

        @pl.when(switched)
        def _():
            pltpu.make_async_copy(w2_hbm.at[0], w2_v, sem2).wait()

        y = lax.dot_general(h, w2_v[...], (((1,), (1,)), ((), ())),
                            preferred_element_type=jnp.float32)
        y_ref[...] = y * w

        nblk = be_ref[jnp.minimum(g + 1, MAX_BLOCKS - 1)]
        e_next2 = _blk_expert(nblk)

        @pl.when((g + 1 < MAX_BLOCKS) & (e_next2 != e) & (nblk != PARK))
        def _():
            pltpu.make_async_copy(w2_hbm.at[e_next2], w2_v, sem2).start()


def _run_ffn(block_expert, x_sorted, w1, w2, router_w):
    grid_spec = pltpu.PrefetchScalarGridSpec(
        num_scalar_prefetch=1,
        grid=(MAX_BLOCKS,),
        in_specs=[
            pl.BlockSpec((B, D_MODEL), lambda g, be: (be[g], 0)),
            pl.BlockSpec(memory_space=pl.ANY),
            pl.BlockSpec(memory_space=pl.ANY),
            pl.BlockSpec((E, D_MODEL), lambda g, be: (0, 0)),
        ],
        out_specs=pl.BlockSpec((B, D_MODEL), lambda g, be: (be[g], 0)),
        scratch_shapes=[
            pltpu.VMEM((2, D_FF, D_MODEL), jnp.float32),
            pltpu.VMEM((D_MODEL, D_FF), jnp.float32),
            pltpu.SMEM((1,), jnp.int32),
            pltpu.SemaphoreType.DMA,
            pltpu.SemaphoreType.DMA,
        ],
    )
    return pl.pallas_call(
        _ffn_body,
        grid_spec=grid_spec,
        out_shape=jax.ShapeDtypeStruct((XS_ROWS, D_MODEL), jnp.float32),
        compiler_params=pltpu.CompilerParams(
            vmem_limit_bytes=100 * 1024 * 1024),
    )(block_expert, x_sorted, w1, w2, router_w)


def _sc_gather(y_sorted, dst2d):
    mesh = plsc.VectorSubcoreMesh(core_axis_name="c", subcore_axis_name="s")

    @functools.partial(
        pl.kernel,
        mesh=mesh,
        out_type=jax.ShapeDtypeStruct((T, D_MODEL), jnp.float32),
        scratch_types=[
            pltpu.VMEM((NCHUNK, CHUNK), jnp.int32),
            pltpu.VMEM((3, CHUNK, D_MODEL), jnp.float32),
            pltpu.SemaphoreType.DMA,
            pltpu.SemaphoreType.DMA,
        ],
    )
    def k(ys_hbm, dst_hbm, out_hbm, idx_v, rows_v, sem_g, sem_s):
        wid = lax.axis_index("s") * NC + lax.axis_index("c")
        pltpu.sync_copy(dst_hbm.at[pl.ds(wid * NCHUNK, NCHUNK)], idx_v)
        base = wid * TPW

        def gy(c):
            return pltpu.make_async_copy(
                ys_hbm.at[idx_v.at[c]], rows_v.at[c % 3], sem_g)

        def st(c):
            return pltpu.make_async_copy(
                rows_v.at[c % 3], out_hbm.at[pl.ds(base + c * CHUNK, CHUNK)],
                sem_s)

        gy(0).start()
        gy(1).start()
        for c in range(NCHUNK):
            gy(c).wait()
            if c + 2 < NCHUNK:
                if c >= 1:
                    st(c - 1).wait()
                gy(c + 2).start()
            st(c).start()
        for c in range(NCHUNK - 3, NCHUNK):
            st(c).wait()

    return k(y_sorted, dst2d)


def kernel(x, router_w, w1, w2):
    dst, xblk = _run_router(x, router_w)
    dst2d = dst.reshape(T // CHUNK, CHUNK)
    x_sorted = _sc_scatter(x, dst2d)
    y_sorted = _run_ffn(xblk, x_sorted, w1, w2, router_w)
    return _sc_gather(y_sorted, dst2d)
